# Initial kernel scaffold; baseline (speedup 1.0000x reference)
import jax
import jax.numpy as jnp
from jax.experimental import pallas as pl


def kernel(x, edge_index, W1, b1, W2, b2):
    raise NotImplementedError("write your pallas kernel here")


# speedup vs baseline: 31.9725x; 31.9725x over previous
import functools

import jax
import jax.numpy as jnp
from jax import lax
from jax.experimental import pallas as pl
from jax.experimental.pallas import tpu as pltpu
from jax.experimental.pallas import tpu_sc as plsc

NC = 2
NS = 16
NW = NC * NS
B = 128


def _deg_body(dst_hbm, ones_hbm, zeros_hbm, degp_hbm, idx_v, ones_v, sptab,
              *, ch, npad):
    c = lax.axis_index("c")
    s = lax.axis_index("s")
    wid = s * NC + c

    @pl.when(s == 0)
    def _zero():
        pltpu.sync_copy(zeros_hbm, sptab)

    pltpu.sync_copy(ones_hbm, ones_v)
    pltpu.sync_copy(dst_hbm.at[wid], idx_v)
    plsc.subcore_barrier()

    def step(j, carry):
        pltpu.sync_copy(ones_v, sptab.at[idx_v.at[j]], add=True)
        return carry

    lax.fori_loop(0, ch, step, 0)
    plsc.subcore_barrier()
    nt = npad // NS
    pltpu.sync_copy(sptab.at[pl.ds(s * nt, nt)],
                    degp_hbm.at[c].at[pl.ds(s * nt, nt)])


def _pass_body(src_hbm, dst_hbm, tab_hbm, zeros_hbm, part_hbm,
               sidx, didx, rows, sem, sptab, *, ch, npad):
    c = lax.axis_index("c")
    s = lax.axis_index("s")
    wid = s * NC + c

    @pl.when(s == 0)
    def _zero():
        pltpu.sync_copy(zeros_hbm, sptab)

    pltpu.sync_copy(src_hbm.at[wid], sidx)
    pltpu.sync_copy(dst_hbm.at[wid], didx)
    plsc.subcore_barrier()

    def step(j, carry):
        pltpu.async_copy(tab_hbm.at[sidx.at[j]], rows, sem).wait()
        pltpu.sync_copy(rows, sptab.at[didx.at[j]], add=True)
        return carry

    lax.fori_loop(0, ch, step, 0)
    plsc.subcore_barrier()
    nt = npad // NS
    pltpu.sync_copy(sptab.at[pl.ds(s * nt, nt)],
                    part_hbm.at[c].at[pl.ds(s * nt, nt)])


def _tc1_body(degp, xp, w1, dis16_o, g1_o):
    dis = lax.rsqrt(degp[0] + degp[1] + 1.0)
    dis16_o[...] = dis
    h = jnp.dot(xp[...], w1[...], preferred_element_type=jnp.float32)
    g1_o[...] = dis * h


def _tc2_body(part, g1, dis16, b1, r_o):
    tmp = part[0] + part[1] + g1[...]
    o = dis16[...] * tmp + b1[...]
    r_o[...] = dis16[...] * jnp.maximum(o, 0.0)


def _tc3_body(part, r, dis16, w2, b2, out_o):
    tmp = dis16[...] * (part[0] + part[1] + r[...])
    z = jnp.dot(tmp, w2[...], preferred_element_type=jnp.float32) + b2[...]
    out_o[...] = jax.nn.softmax(z, axis=-1)


def kernel(x, edge_index, W1, b1, W2, b2):
    n, d = x.shape
    e = edge_index.shape[1]
    h = W1.shape[1]
    cdim = W2.shape[1]
    npad = ((n + 1 + 127) // 128) * 128
    ch = -(-e // (NW * B))
    epad = NW * ch * B

    src = edge_index[0].astype(jnp.int32)
    dst = edge_index[1].astype(jnp.int32)
    pad = epad - e
    src = jnp.concatenate([src, jnp.zeros((pad,), jnp.int32)]).reshape(NW, ch, B)
    dst = jnp.concatenate([dst, jnp.full((pad,), n, jnp.int32)]).reshape(NW, ch, B)

    zeros16 = jnp.zeros((npad, h), jnp.float32)
    ones16 = jnp.ones((B, h), jnp.float32)
    xp = jnp.concatenate([x, jnp.zeros((npad - n, d), jnp.float32)])

    mesh = plsc.VectorSubcoreMesh(core_axis_name="c", subcore_axis_name="s",
                                  num_cores=NC, num_subcores=NS)
    sc_params = pltpu.CompilerParams(use_tc_tiling_on_sc=False)

    deg_call = pl.kernel(
        functools.partial(_deg_body, ch=ch, npad=npad),
        out_type=jax.ShapeDtypeStruct((NC, npad, h), jnp.float32),
        mesh=mesh,
        scratch_types=[
            pltpu.VMEM((ch, B), jnp.int32),
            pltpu.VMEM((B, h), jnp.float32),
            pltpu.VMEM_SHARED((npad, h), jnp.float32),
        ],
        compiler_params=sc_params,
    )
    degp = deg_call(dst, ones16, zeros16)

    pass_call = pl.kernel(
        functools.partial(_pass_body, ch=ch, npad=npad),
        out_type=jax.ShapeDtypeStruct((NC, npad, h), jnp.float32),
        mesh=mesh,
        scratch_types=[
            pltpu.VMEM((ch, B), jnp.int32),
            pltpu.VMEM((ch, B), jnp.int32),
            pltpu.VMEM((B, h), jnp.float32),
            pltpu.SemaphoreType.DMA,
            pltpu.VMEM_SHARED((npad, h), jnp.float32),
        ],
        compiler_params=sc_params,
    )

    dis16, g1 = pl.pallas_call(
        _tc1_body,
        out_shape=[jax.ShapeDtypeStruct((npad, h), jnp.float32)] * 2,
    )(degp, xp, W1)

    part1 = pass_call(src, dst, g1, zeros16)

    r = pl.pallas_call(
        _tc2_body,
        out_shape=jax.ShapeDtypeStruct((npad, h), jnp.float32),
    )(part1, g1, dis16, b1.reshape(1, h))

    part2 = pass_call(src, dst, r, zeros16)

    out = pl.pallas_call(
        _tc3_body,
        out_shape=jax.ShapeDtypeStruct((npad, cdim), jnp.float32),
    )(part2, r, dis16, W2, b2.reshape(1, cdim))

    return out[:n]
